# baseline (speedup 1.0000x reference)
import functools

import jax
import jax.numpy as jnp
from jax import lax
from jax.experimental import pallas as pl
from jax.experimental.pallas import tpu as pltpu
from jax.experimental.pallas import tpu_sc as plsc

_N_USERS = 1340
_N_ITEMS = 733
_BIN = 60
_MAXDAY = 4096
_GMEAN = 4.16275031832388
_B = 16384

_NC = 2
_NS = 16
_L = 16
_NW = _NC * _NS
_BPW = _B // _NW
_G = _BPW // _L
_CH = 4
_GPC = _G // _CH
_RPC = _GPC * _L

_AL_OFF = _N_USERS
_MU_OFF = 2 * _N_USERS
_BI_OFF = 3 * _N_USERS
_W_OFF = 3 * _N_USERS + _N_ITEMS
_WPU_OFF = _W_OFF + 5
_WPI_OFF = _WPU_OFF + 5 * _N_USERS
_BT_OFF = _WPI_OFF + 5 * _N_ITEMS
_TAB_N = _BT_OFF + _MAXDAY + 1

_LOG2_C = (5.0603279522057666e-06, 1.4423955889439901, -0.7169875678731885,
           0.4538582052913859, -0.2723558270407965, 0.11790686115237654,
           -0.024825984443424976)
_POW_SCALE = 0.4 * 0.6931471805599453


@functools.partial(
    pl.kernel,
    out_type=jax.ShapeDtypeStruct((5, _B), jnp.float32),
    mesh=plsc.VectorSubcoreMesh(core_axis_name="c", subcore_axis_name="s",
                                num_cores=_NC, num_subcores=_NS),
    compiler_params=pltpu.CompilerParams(needs_layout_passes=False),
    scratch_types=[
        pltpu.VMEM((5, _BPW), jnp.int32),
        pltpu.VMEM((_TAB_N,), jnp.float32),
        pltpu.VMEM((16,), jnp.float32),
        pltpu.VMEM((128,), jnp.int32),
        pltpu.VMEM((128,), jnp.int32),
        pltpu.VMEM((128,), jnp.int32),
        pltpu.VMEM((128,), jnp.int32),
        pltpu.VMEM((_BPW,), jnp.float32),
        pltpu.VMEM((8, _BPW), jnp.float32),
        pltpu.SemaphoreType.DMA,
        pltpu.SemaphoreType.DMA,
        pltpu.SemaphoreType.DMA,
    ],
)
def _sc_kernel(idx_h, tab_h, bout_h, wbit_h, out_h,
               in_v, tab_t, bo_t,
               widx0, widx1, widx2, widx3, wval_v, out_v, sem, gsem, osem):
    wid = lax.axis_index("s") * _NC + lax.axis_index("c")
    base = wid * _BPW

    tab_cp = pltpu.async_copy(tab_h, tab_t, sem)
    bo_cp = pltpu.async_copy(bout_h, bo_t.at[pl.ds(8, 5)], sem)
    in_cp = pltpu.async_copy(idx_h.at[:, pl.ds(base, _BPW)], in_v, sem)
    with jax.named_scope("in_wait"):
        in_cp.wait()

    iota = lax.iota(jnp.int32, _L)

    widxs = [widx0, widx1, widx2, widx3]
    for k in range(4):
        def widx_body(g, carry, k=k):
            off = k * 128 + g * _L
            it = in_v[1, pl.ds(off, _L)]
            tb = in_v[2, pl.ds(off, _L)]
            widxs[k][pl.ds(g * _L, _L)] = it * _BIN + tb
            return carry

        lax.fori_loop(0, 128 // _L, widx_body, 0)
    gats = [pltpu.async_copy(wbit_h.at[widxs[k]],
                             wval_v.at[pl.ds(k * 128, 128)], gsem)
            for k in range(4)]
    with jax.named_scope("tab_wait"):
        tab_cp.wait()
        bo_cp.wait()
    with jax.named_scope("gat_wait"):
        for c in gats:
            c.wait()

    wvec = [plsc.load_gather(tab_t, [jnp.full((_L,), _W_OFF + j, jnp.int32)])
            for j in range(5)]
    bvec = [plsc.load_gather(bo_t, [jnp.full((_L,), 8 + j, jnp.int32)])
            for j in range(5)]

    def body(g, carry):
        off = g * _L
        u = in_v[0, pl.ds(off, _L)]
        it = in_v[1, pl.ds(off, _L)]
        td = in_v[3, pl.ds(off, _L)]
        mc = in_v[4, pl.ds(off, _L)]

        u5 = _WPU_OFF + u * 5
        i5 = _WPI_OFF + it * 5
        bu = plsc.load_gather(tab_t, [u])
        al = plsc.load_gather(tab_t, [_AL_OFF + u])
        mean = plsc.load_gather(tab_t, [_MU_OFF + u])
        bi = plsc.load_gather(tab_t, [_BI_OFF + it])
        acc = None
        for j in range(5):
            pu = plsc.load_gather(tab_t, [u5 + j])
            pi = plsc.load_gather(tab_t, [i5 + j])
            acc = pu * pi if acc is None else acc + pu * pi
        wbitv = wval_v[pl.ds(off, _L)]
        btv = plsc.load_gather(tab_t, [_BT_OFF + mc])

        tdf = td.astype(jnp.float32) - mean
        d = jnp.abs(tdf)
        bits = plsc.bitcast(d, jnp.int32)
        e = ((bits >> 23) - 127).astype(jnp.float32)
        m = plsc.bitcast((bits & 0x007FFFFF) | 0x3F800000, jnp.float32)
        t = m - 1.0
        p = jnp.float32(_LOG2_C[6])
        for c_ in _LOG2_C[5::-1]:
            p = p * t + jnp.float32(c_)
        dev_mag = jnp.exp((e + p) * jnp.float32(_POW_SCALE))
        dev = jnp.sign(tdf) * dev_mag

        pred = _GMEAN + bu + al * dev + btv + bi + wbitv + acc

        for j in range(5):
            out_v[j, pl.ds(off, _L)] = pred * wvec[j] + bvec[j]
        return carry

    with jax.named_scope("main_loop"):
        for k in range(_CH):
            lax.fori_loop(k * _GPC, (k + 1) * _GPC, body, 0)
            pltpu.async_copy(
                out_v.at[pl.ds(0, 5), pl.ds(k * _RPC, _RPC)],
                out_h.at[:, pl.ds(base + k * _RPC, _RPC)],
                osem)

    with jax.named_scope("out_wait"):
        for k in range(_CH):
            pltpu.make_async_copy(
                out_v.at[pl.ds(0, 5), pl.ds(k * _RPC, _RPC)],
                out_h.at[:, pl.ds(base + k * _RPC, _RPC)],
                osem).wait()


def kernel(user_ids, item_ids, itbin, tday, maxday_cat, mean_ud,
           BU, BI, WPU, WPI, WBIT, Alpha, BTDay, W_out, b_out):
    f32 = jnp.float32
    i32 = jnp.int32
    cat1 = jnp.concatenate(
        [BU, Alpha, mean_ud.astype(f32), BI, W_out], axis=0).reshape(-1)
    wp = jnp.concatenate([WPU, WPI], axis=0).reshape(-1)
    tab = jnp.concatenate([cat1, wp, BTDay])
    idx = jnp.stack([user_ids.astype(i32), item_ids.astype(i32),
                     itbin.astype(i32), tday.astype(i32),
                     maxday_cat.astype(i32)])
    out = _sc_kernel(idx, tab, b_out, WBIT.reshape(-1))
    return out.T


# speedup vs baseline: 1.0102x; 1.0102x over previous
import functools

import jax
import jax.numpy as jnp
from jax import lax
from jax.experimental import pallas as pl
from jax.experimental.pallas import tpu as pltpu
from jax.experimental.pallas import tpu_sc as plsc

_N_USERS = 1340
_N_ITEMS = 733
_BIN = 60
_MAXDAY = 4096
_GMEAN = 4.16275031832388
_B = 16384

_NC = 2
_NS = 16
_L = 16
_NW = _NC * _NS
_BPW = _B // _NW
_G = _BPW // _L
_CH = 4
_GPC = _G // _CH
_RPC = _GPC * _L

_AL_OFF = _N_USERS
_MU_OFF = 2 * _N_USERS
_BI_OFF = 3 * _N_USERS
_W_OFF = 3 * _N_USERS + _N_ITEMS
_WPU_OFF = _W_OFF + 5
_WPI_OFF = _WPU_OFF + 5 * _N_USERS
_BT_OFF = _WPI_OFF + 5 * _N_ITEMS
_TAB_N = _BT_OFF + _MAXDAY + 1

_LOG2_C = (5.0603279522057666e-06, 1.4423955889439901, -0.7169875678731885,
           0.4538582052913859, -0.2723558270407965, 0.11790686115237654,
           -0.024825984443424976)
_POW_SCALE = 0.4 * 0.6931471805599453


@functools.partial(
    pl.kernel,
    out_type=jax.ShapeDtypeStruct((5, _B), jnp.float32),
    mesh=plsc.VectorSubcoreMesh(core_axis_name="c", subcore_axis_name="s",
                                num_cores=_NC, num_subcores=_NS),
    compiler_params=pltpu.CompilerParams(needs_layout_passes=False),
    scratch_types=[
        pltpu.VMEM((_BPW,), jnp.int32),
        pltpu.VMEM((_BPW,), jnp.int32),
        pltpu.VMEM((_BPW,), jnp.int32),
        pltpu.VMEM((_BPW,), jnp.int32),
        pltpu.VMEM((_BPW,), jnp.int32),
        pltpu.VMEM((_TAB_N,), jnp.float32),
        pltpu.VMEM((16,), jnp.float32),
        pltpu.VMEM((128,), jnp.int32),
        pltpu.VMEM((128,), jnp.int32),
        pltpu.VMEM((128,), jnp.int32),
        pltpu.VMEM((128,), jnp.int32),
        pltpu.VMEM((_BPW,), jnp.float32),
        pltpu.VMEM((8, _BPW), jnp.float32),
        pltpu.SemaphoreType.DMA,
        pltpu.SemaphoreType.DMA,
        pltpu.SemaphoreType.DMA,
    ],
)
def _sc_kernel(uids_h, iids_h, itbin_h, tday_h, mcat_h, tab_h, bout_h,
               wbit_h, out_h,
               u_v, i_v, tb_v, td_v, mc_v, tab_t, bo_t,
               widx0, widx1, widx2, widx3, wval_v, out_v, sem, gsem, osem):
    wid = lax.axis_index("s") * _NC + lax.axis_index("c")
    base = wid * _BPW

    tab_cp = pltpu.async_copy(tab_h, tab_t, sem)
    bo_cp = pltpu.async_copy(bout_h, bo_t.at[pl.ds(8, 5)], sem)
    in_cps = [
        pltpu.async_copy(uids_h.at[pl.ds(base, _BPW)], u_v, sem),
        pltpu.async_copy(iids_h.at[pl.ds(base, _BPW)], i_v, sem),
        pltpu.async_copy(itbin_h.at[pl.ds(base, _BPW)], tb_v, sem),
        pltpu.async_copy(tday_h.at[pl.ds(base, _BPW)], td_v, sem),
        pltpu.async_copy(mcat_h.at[pl.ds(base, _BPW)], mc_v, sem),
    ]
    with jax.named_scope("in_wait"):
        for c in in_cps:
            c.wait()

    iota = lax.iota(jnp.int32, _L)

    widxs = [widx0, widx1, widx2, widx3]
    for k in range(4):
        def widx_body(g, carry, k=k):
            off = k * 128 + g * _L
            it = i_v[pl.ds(off, _L)]
            tb = tb_v[pl.ds(off, _L)]
            widxs[k][pl.ds(g * _L, _L)] = it * _BIN + tb
            return carry

        lax.fori_loop(0, 128 // _L, widx_body, 0)
    gats = [pltpu.async_copy(wbit_h.at[widxs[k]],
                             wval_v.at[pl.ds(k * 128, 128)], gsem)
            for k in range(4)]
    with jax.named_scope("tab_wait"):
        tab_cp.wait()
        bo_cp.wait()
    with jax.named_scope("gat_wait"):
        for c in gats:
            c.wait()

    wvec = [plsc.load_gather(tab_t, [jnp.full((_L,), _W_OFF + j, jnp.int32)])
            for j in range(5)]
    bvec = [plsc.load_gather(bo_t, [jnp.full((_L,), 8 + j, jnp.int32)])
            for j in range(5)]

    def body(g, carry):
        off = g * _L
        u = u_v[pl.ds(off, _L)]
        it = i_v[pl.ds(off, _L)]
        td = td_v[pl.ds(off, _L)]
        mc = mc_v[pl.ds(off, _L)]

        u5 = _WPU_OFF + u * 5
        i5 = _WPI_OFF + it * 5
        bu = plsc.load_gather(tab_t, [u])
        al = plsc.load_gather(tab_t, [_AL_OFF + u])
        mean = plsc.load_gather(tab_t, [_MU_OFF + u])
        bi = plsc.load_gather(tab_t, [_BI_OFF + it])
        acc = None
        for j in range(5):
            pu = plsc.load_gather(tab_t, [u5 + j])
            pi = plsc.load_gather(tab_t, [i5 + j])
            acc = pu * pi if acc is None else acc + pu * pi
        wbitv = wval_v[pl.ds(off, _L)]
        btv = plsc.load_gather(tab_t, [_BT_OFF + mc])

        tdf = td.astype(jnp.float32) - mean
        d = jnp.abs(tdf)
        bits = plsc.bitcast(d, jnp.int32)
        e = ((bits >> 23) - 127).astype(jnp.float32)
        m = plsc.bitcast((bits & 0x007FFFFF) | 0x3F800000, jnp.float32)
        t = m - 1.0
        p = jnp.float32(_LOG2_C[6])
        for c_ in _LOG2_C[5::-1]:
            p = p * t + jnp.float32(c_)
        dev_mag = jnp.exp((e + p) * jnp.float32(_POW_SCALE))
        dev = jnp.sign(tdf) * dev_mag

        pred = _GMEAN + bu + al * dev + btv + bi + wbitv + acc

        for j in range(5):
            out_v[j, pl.ds(off, _L)] = pred * wvec[j] + bvec[j]
        return carry

    with jax.named_scope("main_loop"):
        for k in range(_CH):
            lax.fori_loop(k * _GPC, (k + 1) * _GPC, body, 0)
            pltpu.async_copy(
                out_v.at[pl.ds(0, 5), pl.ds(k * _RPC, _RPC)],
                out_h.at[:, pl.ds(base + k * _RPC, _RPC)],
                osem)

    with jax.named_scope("out_wait"):
        for k in range(_CH):
            pltpu.make_async_copy(
                out_v.at[pl.ds(0, 5), pl.ds(k * _RPC, _RPC)],
                out_h.at[:, pl.ds(base + k * _RPC, _RPC)],
                osem).wait()


def kernel(user_ids, item_ids, itbin, tday, maxday_cat, mean_ud,
           BU, BI, WPU, WPI, WBIT, Alpha, BTDay, W_out, b_out):
    f32 = jnp.float32
    i32 = jnp.int32
    cat1 = jnp.concatenate(
        [BU, Alpha, mean_ud.astype(f32), BI, W_out], axis=0).reshape(-1)
    wp = jnp.concatenate([WPU, WPI], axis=0).reshape(-1)
    tab = jnp.concatenate([cat1, wp, BTDay])
    out = _sc_kernel(user_ids.astype(i32), item_ids.astype(i32),
                     itbin.astype(i32), tday.astype(i32),
                     maxday_cat.astype(i32), tab, b_out, WBIT.reshape(-1))
    return out.T
